# baseline (speedup 1.0000x reference)
import functools

import jax
import jax.numpy as jnp
from jax import lax
from jax.experimental import pallas as pl
from jax.experimental.pallas import tpu as pltpu
from jax.experimental.pallas import tpu_sc as plsc

VOCAB = 1000000
BATCH = 16384
DIM = 64

_NC = 2
_NS = 16
_NW = _NC * _NS
_BPW = BATCH // _NW
_CH = 32
_NCHUNK = _BPW // _CH

_mesh = plsc.VectorSubcoreMesh(core_axis_name="c", subcore_axis_name="s")


@functools.partial(
    pl.kernel,
    mesh=_mesh,
    out_type=[
        jax.ShapeDtypeStruct((BATCH, DIM), jnp.float32),
        jax.ShapeDtypeStruct((BATCH, DIM), jnp.float32),
    ],
    scratch_types=[
        pltpu.VMEM((_BPW,), jnp.int32),
        pltpu.VMEM((_BPW, DIM), jnp.float32),
        pltpu.SemaphoreType.DMA,
        pltpu.SemaphoreType.DMA,
    ],
)
def _sc_gather(x0_hbm, x1_hbm, idx_hbm, g0_hbm, g1_hbm, idx_v, rows_v,
               sem_s, sem_d):
    wid = lax.axis_index("s") * _NC + lax.axis_index("c")
    base = wid * _BPW
    pltpu.sync_copy(idx_hbm.at[pl.ds(base, _BPW)], idx_v)

    def issue_chunk(c, carry):
        off = c * 16
        vec = idx_v[pl.ds(off, 16)]
        for k in range(16):
            i = vec[k]
            pltpu.make_async_copy(
                x0_hbm.at[pl.ds(i, 1), :],
                rows_v.at[pl.ds(off + k, 1), :],
                sem_s,
            ).start()
            pltpu.make_async_copy(
                x1_hbm.at[pl.ds(i, 1), :],
                g1_hbm.at[pl.ds(base + off + k, 1), :],
                sem_d,
            ).start()
        return carry

    lax.fori_loop(0, _BPW // 16, issue_chunk, 0)
    pltpu.make_async_copy(
        x0_hbm.at[pl.ds(0, _BPW), :], rows_v, sem_s
    ).wait()
    pltpu.sync_copy(rows_v, g0_hbm.at[pl.ds(base, _BPW), :])
    pltpu.make_async_copy(
        x1_hbm.at[pl.ds(0, _BPW), :], g1_hbm.at[pl.ds(base, _BPW), :], sem_d
    ).wait()


_BM = 1024


def _mm_body(g0_ref, g1_ref, w0_ref, w1_ref, o_ref):
    o_ref[...] = (
        jnp.dot(g0_ref[...], w0_ref[...], preferred_element_type=jnp.float32)
        + jnp.dot(g1_ref[...], w1_ref[...], preferred_element_type=jnp.float32)
    )


_mm = pl.pallas_call(
    _mm_body,
    grid=(BATCH // _BM,),
    in_specs=[
        pl.BlockSpec((_BM, DIM), lambda i: (i, 0)),
        pl.BlockSpec((_BM, DIM), lambda i: (i, 0)),
        pl.BlockSpec((DIM, DIM), lambda i: (0, 0)),
        pl.BlockSpec((DIM, DIM), lambda i: (0, 0)),
    ],
    out_specs=pl.BlockSpec((_BM, DIM), lambda i: (i, 0)),
    out_shape=jax.ShapeDtypeStruct((BATCH, DIM), jnp.float32),
)


def kernel(x0, x1, W, indices):
    g0, g1 = _sc_gather(x0, x1, indices)
    return _mm(g0, g1, W[:DIM], W[DIM:])


# speedup vs baseline: 1.3412x; 1.3412x over previous
import functools

import jax
import jax.numpy as jnp
from jax import lax
from jax.experimental import pallas as pl
from jax.experimental.pallas import tpu as pltpu
from jax.experimental.pallas import tpu_sc as plsc

VOCAB = 1000000
BATCH = 16384
DIM = 64

_NC = 2
_NS = 16
_NW = _NC * _NS
_BPW = BATCH // _NW
_CH = 32
_NCHUNK = _BPW // _CH

_mesh = plsc.VectorSubcoreMesh(core_axis_name="c", subcore_axis_name="s")


@functools.partial(
    pl.kernel,
    mesh=_mesh,
    out_type=[
        jax.ShapeDtypeStruct((BATCH, DIM), jnp.float32),
        jax.ShapeDtypeStruct((BATCH, DIM), jnp.float32),
    ],
    scratch_types=[
        pltpu.VMEM((_BPW,), jnp.int32),
        pltpu.VMEM((_BPW, DIM), jnp.float32),
        pltpu.SemaphoreType.DMA((8,)),
    ],
)
def _sc_gather(x0_hbm, x1_hbm, idx_hbm, g0_hbm, g1_hbm, idx_v, rows_v, sems):
    wid = lax.axis_index("s") * _NC + lax.axis_index("c")
    base = wid * _BPW
    pltpu.sync_copy(idx_hbm.at[pl.ds(base, _BPW)], idx_v)

    def one_table(x_hbm, g_hbm):
        def issue_chunk(c, carry):
            off = c * 16
            vec = idx_v[pl.ds(off, 16)]
            for k in range(16):
                i = vec[k]
                pltpu.make_async_copy(
                    x_hbm.at[pl.ds(i, 1), :],
                    rows_v.at[pl.ds(off + k, 1), :],
                    sems.at[k % 8],
                ).start()
            return carry

        lax.fori_loop(0, _BPW // 16, issue_chunk, 0)
        for s in range(8):
            pltpu.make_async_copy(
                x_hbm.at[pl.ds(0, _BPW // 8), :],
                rows_v.at[pl.ds(s * (_BPW // 8), _BPW // 8), :],
                sems.at[s],
            ).wait()
        pltpu.sync_copy(rows_v, g_hbm.at[pl.ds(base, _BPW), :])

    one_table(x0_hbm, g0_hbm)
    one_table(x1_hbm, g1_hbm)


_BM = 1024


def _mm_body(g0_ref, g1_ref, w0_ref, w1_ref, o_ref):
    o_ref[...] = (
        jnp.dot(g0_ref[...], w0_ref[...], preferred_element_type=jnp.float32)
        + jnp.dot(g1_ref[...], w1_ref[...], preferred_element_type=jnp.float32)
    )


_mm = pl.pallas_call(
    _mm_body,
    grid=(BATCH // _BM,),
    in_specs=[
        pl.BlockSpec((_BM, DIM), lambda i: (i, 0)),
        pl.BlockSpec((_BM, DIM), lambda i: (i, 0)),
        pl.BlockSpec((DIM, DIM), lambda i: (0, 0)),
        pl.BlockSpec((DIM, DIM), lambda i: (0, 0)),
    ],
    out_specs=pl.BlockSpec((_BM, DIM), lambda i: (i, 0)),
    out_shape=jax.ShapeDtypeStruct((BATCH, DIM), jnp.float32),
)


def kernel(x0, x1, W, indices):
    g0, g1 = _sc_gather(x0, x1, indices)
    return _mm(g0, g1, W[:DIM], W[DIM:])


# speedup vs baseline: 2.0011x; 1.4921x over previous
import functools

import jax
import jax.numpy as jnp
from jax import lax
from jax.experimental import pallas as pl
from jax.experimental.pallas import tpu as pltpu
from jax.experimental.pallas import tpu_sc as plsc

VOCAB = 1000000
BATCH = 16384
DIM = 64

_NC = 2
_NS = 16
_NW = _NC * _NS
_BPW = BATCH // _NW

_mesh = plsc.VectorSubcoreMesh(core_axis_name="c", subcore_axis_name="s")


@functools.partial(
    pl.kernel,
    mesh=_mesh,
    out_type=jax.ShapeDtypeStruct((BATCH, 2 * DIM), jnp.float32),
    scratch_types=[
        pltpu.VMEM((_BPW,), jnp.int32),
        pltpu.VMEM((_BPW, 2 * DIM), jnp.float32),
        pltpu.SemaphoreType.DMA,
    ],
)
def _sc_gather(xc_hbm, idx_hbm, h_hbm, idx_v, rows_v, sem):
    wid = lax.axis_index("s") * _NC + lax.axis_index("c")
    base = wid * _BPW
    pltpu.sync_copy(idx_hbm.at[pl.ds(base, _BPW)], idx_v)

    def issue_chunk(c, carry):
        off = c * 16
        vec = idx_v[pl.ds(off, 16)]
        for k in range(16):
            i = vec[k]
            pltpu.make_async_copy(
                xc_hbm.at[pl.ds(i, 1), :],
                rows_v.at[pl.ds(off + k, 1), :],
                sem,
            ).start()
        return carry

    lax.fori_loop(0, _BPW // 16, issue_chunk, 0)
    pltpu.make_async_copy(xc_hbm.at[pl.ds(0, _BPW), :], rows_v, sem).wait()
    pltpu.sync_copy(rows_v, h_hbm.at[pl.ds(base, _BPW), :])


_BT = 8192


def _tr_body(x0t_ref, x1t_ref, o_ref):
    o_ref[...] = jnp.concatenate([x0t_ref[...].T, x1t_ref[...].T], axis=1)


_transpose = pl.pallas_call(
    _tr_body,
    grid=(VOCAB // _BT + 1,),
    in_specs=[
        pl.BlockSpec((DIM, _BT), lambda i: (0, i)),
        pl.BlockSpec((DIM, _BT), lambda i: (0, i)),
    ],
    out_specs=pl.BlockSpec((_BT, 2 * DIM), lambda i: (i, 0)),
    out_shape=jax.ShapeDtypeStruct((VOCAB, 2 * DIM), jnp.float32),
)


_BM = 1024


def _mm_body(h_ref, w_ref, o_ref):
    o_ref[...] = jnp.dot(
        h_ref[...], w_ref[...], preferred_element_type=jnp.float32
    )


_mm = pl.pallas_call(
    _mm_body,
    grid=(BATCH // _BM,),
    in_specs=[
        pl.BlockSpec((_BM, 2 * DIM), lambda i: (i, 0)),
        pl.BlockSpec((2 * DIM, DIM), lambda i: (0, 0)),
    ],
    out_specs=pl.BlockSpec((_BM, DIM), lambda i: (i, 0)),
    out_shape=jax.ShapeDtypeStruct((BATCH, DIM), jnp.float32),
)


def kernel(x0, x1, W, indices):
    xc = _transpose(x0.T, x1.T)
    h = _sc_gather(xc, indices)
    return _mm(h, W)


# speedup vs baseline: 2.6075x; 1.3030x over previous
import functools

import jax
import jax.numpy as jnp
from jax import lax
from jax.experimental import pallas as pl
from jax.experimental.pallas import tpu as pltpu
from jax.experimental.pallas import tpu_sc as plsc

VOCAB = 1000000
BATCH = 16384
DIM = 64

_NC = 2
_NS = 16
_NW = _NC * _NS
_BPW = BATCH // _NW

_mesh = plsc.VectorSubcoreMesh(core_axis_name="c", subcore_axis_name="s")


@functools.partial(
    pl.kernel,
    mesh=_mesh,
    out_type=jax.ShapeDtypeStruct((BATCH, 2 * DIM), jnp.float32),
    scratch_types=[
        pltpu.VMEM((_BPW,), jnp.int32),
        pltpu.VMEM((_BPW, 2 * DIM), jnp.float32),
        pltpu.SemaphoreType.DMA,
    ],
)
def _sc_gather(xc_hbm, idx_hbm, h_hbm, idx_v, rows_v, sem):
    wid = lax.axis_index("s") * _NC + lax.axis_index("c")
    base = wid * _BPW
    pltpu.sync_copy(idx_hbm.at[pl.ds(base, _BPW)], idx_v)

    def issue_chunk(c, carry):
        off = c * 16
        vec = idx_v[pl.ds(off, 16)]
        for k in range(16):
            i = vec[k]
            pltpu.make_async_copy(
                xc_hbm.at[pl.ds(i, 1), :],
                rows_v.at[pl.ds(off + k, 1), :],
                sem,
            ).start()
        return carry

    lax.fori_loop(0, _BPW // 16, issue_chunk, 0)
    pltpu.make_async_copy(xc_hbm.at[pl.ds(0, _BPW), :], rows_v, sem).wait()
    pltpu.sync_copy(rows_v, h_hbm.at[pl.ds(base, _BPW), :])


_BT = 8192


def _tr_body(x0t_ref, x1t_ref, o_ref):
    o_ref[...] = jnp.concatenate([x0t_ref[...], x1t_ref[...]], axis=0).T


_transpose = pl.pallas_call(
    _tr_body,
    grid=(VOCAB // _BT + 1,),
    in_specs=[
        pl.BlockSpec((DIM, _BT), lambda i: (0, i)),
        pl.BlockSpec((DIM, _BT), lambda i: (0, i)),
    ],
    out_specs=pl.BlockSpec((_BT, 2 * DIM), lambda i: (i, 0)),
    out_shape=jax.ShapeDtypeStruct((VOCAB, 2 * DIM), jnp.float32),
)


_BM = 1024


def _mm_body(h_ref, w_ref, o_ref):
    o_ref[...] = jnp.dot(
        h_ref[...], w_ref[...], preferred_element_type=jnp.float32
    )


_mm = pl.pallas_call(
    _mm_body,
    grid=(BATCH // _BM,),
    in_specs=[
        pl.BlockSpec((_BM, 2 * DIM), lambda i: (i, 0)),
        pl.BlockSpec((2 * DIM, DIM), lambda i: (0, 0)),
    ],
    out_specs=pl.BlockSpec((_BM, DIM), lambda i: (i, 0)),
    out_shape=jax.ShapeDtypeStruct((BATCH, DIM), jnp.float32),
)


def kernel(x0, x1, W, indices):
    xc = _transpose(x0.T, x1.T)
    h = _sc_gather(xc, indices)
    return _mm(h, W)


# speedup vs baseline: 2.6847x; 1.0296x over previous
import functools

import jax
import jax.numpy as jnp
from jax import lax
from jax.experimental import pallas as pl
from jax.experimental.pallas import tpu as pltpu
from jax.experimental.pallas import tpu_sc as plsc

VOCAB = 1000000
BATCH = 16384
DIM = 64

_NC = 2
_NS = 16
_NW = _NC * _NS
_BPW = BATCH // _NW

_mesh = plsc.VectorSubcoreMesh(core_axis_name="c", subcore_axis_name="s")


@functools.partial(
    pl.kernel,
    mesh=_mesh,
    out_type=jax.ShapeDtypeStruct((BATCH, 2 * DIM), jnp.float32),
    scratch_types=[
        pltpu.VMEM((_BPW,), jnp.int32),
        pltpu.VMEM((_BPW, 2 * DIM), jnp.float32),
        pltpu.SemaphoreType.DMA,
    ],
)
def _sc_gather(xc_hbm, idx_hbm, h_hbm, idx_v, rows_v, sem):
    wid = lax.axis_index("s") * _NC + lax.axis_index("c")
    base = wid * _BPW
    pltpu.sync_copy(idx_hbm.at[pl.ds(base, _BPW)], idx_v)

    def issue_chunk(c, carry):
        off = c * 16
        vec = idx_v[pl.ds(off, 16)]
        for k in range(16):
            i = vec[k]
            pltpu.make_async_copy(
                xc_hbm.at[pl.ds(i, 1), :],
                rows_v.at[pl.ds(off + k, 1), :],
                sem,
            ).start()
        return carry

    lax.fori_loop(0, _BPW // 16, issue_chunk, 0)
    pltpu.make_async_copy(xc_hbm.at[pl.ds(0, _BPW), :], rows_v, sem).wait()
    pltpu.sync_copy(rows_v, h_hbm.at[pl.ds(base, _BPW), :])


_BT = 16384


def _tr_body(x0t_ref, x1t_ref, o_ref):
    o_ref[...] = jnp.concatenate([x0t_ref[...], x1t_ref[...]], axis=0).T


_transpose = pl.pallas_call(
    _tr_body,
    grid=(VOCAB // _BT + 1,),
    in_specs=[
        pl.BlockSpec((DIM, _BT), lambda i: (0, i)),
        pl.BlockSpec((DIM, _BT), lambda i: (0, i)),
    ],
    out_specs=pl.BlockSpec((_BT, 2 * DIM), lambda i: (i, 0)),
    out_shape=jax.ShapeDtypeStruct((VOCAB, 2 * DIM), jnp.float32),
)


_BM = 1024


def _mm_body(h_ref, w_ref, o_ref):
    o_ref[...] = jnp.dot(
        h_ref[...], w_ref[...], preferred_element_type=jnp.float32
    )


_mm = pl.pallas_call(
    _mm_body,
    grid=(BATCH // _BM,),
    in_specs=[
        pl.BlockSpec((_BM, 2 * DIM), lambda i: (i, 0)),
        pl.BlockSpec((2 * DIM, DIM), lambda i: (0, 0)),
    ],
    out_specs=pl.BlockSpec((_BM, DIM), lambda i: (i, 0)),
    out_shape=jax.ShapeDtypeStruct((BATCH, DIM), jnp.float32),
)


def kernel(x0, x1, W, indices):
    xc = _transpose(x0.T, x1.T)
    h = _sc_gather(xc, indices)
    return _mm(h, W)


# speedup vs baseline: 2.7401x; 1.0206x over previous
import functools

import jax
import jax.numpy as jnp
from jax import lax
from jax.experimental import pallas as pl
from jax.experimental.pallas import tpu as pltpu
from jax.experimental.pallas import tpu_sc as plsc

VOCAB = 1000000
BATCH = 16384
DIM = 64

_NC = 2
_NS = 16
_NW = _NC * _NS
_BPW = BATCH // _NW

_mesh = plsc.VectorSubcoreMesh(core_axis_name="c", subcore_axis_name="s")


@functools.partial(
    pl.kernel,
    mesh=_mesh,
    out_type=jax.ShapeDtypeStruct((BATCH, 2 * DIM), jnp.float32),
    scratch_types=[
        pltpu.VMEM((_BPW,), jnp.int32),
        pltpu.VMEM((_BPW, 2 * DIM), jnp.float32),
        pltpu.SemaphoreType.DMA,
    ],
)
def _sc_gather(xc_hbm, idx_hbm, h_hbm, idx_v, rows_v, sem):
    wid = lax.axis_index("s") * _NC + lax.axis_index("c")
    base = wid * _BPW
    pltpu.sync_copy(idx_hbm.at[pl.ds(base, _BPW)], idx_v)

    def issue_chunk(c, carry):
        off = c * 16
        vec = idx_v[pl.ds(off, 16)]
        for k in range(16):
            i = vec[k]
            pltpu.make_async_copy(
                xc_hbm.at[pl.ds(i, 1), :],
                rows_v.at[pl.ds(off + k, 1), :],
                sem,
            ).start()
        return carry

    lax.fori_loop(0, _BPW // 16, issue_chunk, 0)
    pltpu.make_async_copy(xc_hbm.at[pl.ds(0, _BPW), :], rows_v, sem).wait()
    pltpu.sync_copy(rows_v, h_hbm.at[pl.ds(base, _BPW), :])


_BT = 16384


def _tr_body(x0t_ref, x1t_ref, o_ref):
    o_ref[...] = jnp.concatenate([x0t_ref[...], x1t_ref[...]], axis=0).T


_transpose = pl.pallas_call(
    _tr_body,
    grid=(VOCAB // _BT + 1,),
    in_specs=[
        pl.BlockSpec((DIM, _BT), lambda i: (0, i)),
        pl.BlockSpec((DIM, _BT), lambda i: (0, i)),
    ],
    out_specs=pl.BlockSpec((_BT, 2 * DIM), lambda i: (i, 0)),
    out_shape=jax.ShapeDtypeStruct((VOCAB, 2 * DIM), jnp.float32),
)


_BM = 1024


def _mm_body(h_ref, w_ref, o_ref):
    o_ref[...] = lax.dot_general(
        w_ref[...], h_ref[...],
        dimension_numbers=(((0,), (1,)), ((), ())),
        preferred_element_type=jnp.float32,
    )


_mm = pl.pallas_call(
    _mm_body,
    grid=(BATCH // _BM,),
    in_specs=[
        pl.BlockSpec((_BM, 2 * DIM), lambda i: (i, 0)),
        pl.BlockSpec((2 * DIM, DIM), lambda i: (0, 0)),
    ],
    out_specs=pl.BlockSpec((DIM, _BM), lambda i: (0, i)),
    out_shape=jax.ShapeDtypeStruct((DIM, BATCH), jnp.float32),
)


def kernel(x0, x1, W, indices):
    xc = _transpose(x0.T, x1.T)
    h = _sc_gather(xc, indices)
    return _mm(h, W).T
